# baseline (speedup 1.0000x reference)
import functools
import math

import jax
import jax.numpy as jnp
from jax import lax
from jax.experimental import pallas as pl
from jax.experimental.pallas import tpu as pltpu
from jax.experimental.pallas import tpu_sc as plsc

_NUM_CORES = 2
_NUM_SUBCORES = 16
_NUM_WORKERS = _NUM_CORES * _NUM_SUBCORES
_LANES = 16
_PG = 4
_CG = 8
_P = 64
_NBUF = 4


@functools.partial(jax.jit, static_argnames=("pad",))
def _sc_lpe(xf, mk, idx, pe_table, pad):
    n = xf.shape[0]
    dc = xf.shape[2]
    d = _CG * dc
    v = pe_table.shape[1] // dc
    scale = math.sqrt(d)
    per_pg = n // _PG
    n_chunks = per_pg // _P
    assert n_chunks % _NBUF == 0
    mesh = plsc.VectorSubcoreMesh(core_axis_name="c", subcore_axis_name="s")

    @functools.partial(
        pl.kernel,
        mesh=mesh,
        out_type=jax.ShapeDtypeStruct((n, _CG, dc), jnp.float32),
        compiler_params=pltpu.CompilerParams(needs_layout_passes=False),
        scratch_types=[
            pltpu.VMEM((v * dc,), jnp.float32),
            *[pltpu.VMEM((_P,), jnp.int32) for _ in range(_NBUF)],
            *[pltpu.VMEM((_P,), jnp.int32) for _ in range(_NBUF)],
            *[pltpu.VMEM((_P, dc), jnp.float32) for _ in range(_NBUF)],
            *[pltpu.SemaphoreType.DMA for _ in range(2 * _NBUF)],
        ],
    )
    def k(x_hbm, mk_hbm, idx_hbm, tab_hbm, out_hbm, tab_v, *bufs):
        idxb = bufs[0:_NBUF]
        mkb = bufs[_NBUF : 2 * _NBUF]
        xb = bufs[2 * _NBUF : 3 * _NBUF]
        sin = bufs[3 * _NBUF : 4 * _NBUF]
        sout = bufs[4 * _NBUF : 5 * _NBUF]
        wid = lax.axis_index("s") * _NUM_CORES + lax.axis_index("c")
        pg = wid % _PG
        cg = wid // _PG
        col0 = cg * dc
        pos0 = pg * per_pg

        pltpu.sync_copy(tab_hbm.at[cg], tab_v)

        def issue_in(c, b):
            base = pos0 + c * _P
            pltpu.async_copy(idx_hbm.at[pl.ds(base, _P)], idxb[b], sin[b])
            pltpu.async_copy(mk_hbm.at[pl.ds(base, _P)], mkb[b], sin[b])
            pltpu.async_copy(
                x_hbm.at[pl.ds(base, _P), cg, :], xb[b], sin[b]
            )

        def wait_in(c, b):
            base = pos0 + c * _P
            pltpu.make_async_copy(idx_hbm.at[pl.ds(base, _P)], idxb[b], sin[b]).wait()
            pltpu.make_async_copy(mk_hbm.at[pl.ds(base, _P)], mkb[b], sin[b]).wait()
            pltpu.make_async_copy(
                x_hbm.at[pl.ds(base, _P), cg, :], xb[b], sin[b]
            ).wait()

        def issue_out(c, b):
            base = pos0 + c * _P
            pltpu.async_copy(
                xb[b], out_hbm.at[pl.ds(base, _P), cg, :], sout[b]
            )

        def wait_out(c, b):
            base = pos0 + c * _P
            pltpu.make_async_copy(
                xb[b], out_hbm.at[pl.ds(base, _P), cg, :], sout[b]
            ).wait()

        iota16 = lax.iota(jnp.int32, _LANES)

        for b in range(_NBUF - 1):
            issue_in(b, b)

        @pl.loop(0, n_chunks, step=_NBUF)
        def _main(g):
            for b in range(_NBUF):
                c = g + b
                wait_in(c, b)

                def _pad(i, carry):
                    sl = pl.ds(i * _LANES, _LANES)
                    idxb[b][sl] = jnp.where(
                        mkb[b][sl] != 0, pad, jnp.minimum(idxb[b][sl], pad)
                    )
                    return carry

                lax.fori_loop(0, _P // _LANES, _pad, 0)

                def _gfma(g2, carry):
                    pb = g2 * _LANES
                    vec = idxb[b][pl.ds(pb, _LANES)]
                    vecx = vec * dc
                    posvec = pb + iota16
                    for j in range(dc):
                        jv = jnp.full((_LANES,), j, jnp.int32)
                        t = plsc.load_gather(tab_v, [vecx + j])
                        xv = plsc.load_gather(xb[b], [posvec, jv])
                        plsc.store_scatter(xb[b], [posvec, jv], scale * xv + t)
                    return carry

                lax.fori_loop(0, _P // _LANES, _gfma, 0)

                issue_out(c, b)
                nxt = c + _NBUF - 1
                bp = (b + _NBUF - 1) % _NBUF

                @pl.when(nxt < n_chunks)
                def _():
                    @pl.when(c >= 1)
                    def _():
                        wait_out(c - 1, bp)

                    issue_in(nxt, bp)

        for b in range(_NBUF):
            wait_out(n_chunks - _NBUF + b, b)

    return k(xf, mk, idx, pe_table)


def kernel(x, mask, indices, pe_table):
    b, s, d = x.shape
    n = b * s
    v = pe_table.shape[0]
    dc = d // _CG
    xf = x.reshape(n, _CG, dc)
    tab = pe_table.reshape(v, _CG, dc).transpose(1, 0, 2).reshape(_CG, v * dc)
    mk = mask.reshape(n).astype(jnp.int32)
    idx = indices.reshape(n).astype(jnp.int32)
    out = _sc_lpe(xf, mk, idx, tab, v - 1)
    return out.reshape(b, s, d)


# speedup vs baseline: 1.4795x; 1.4795x over previous
import functools
import math

import jax
import jax.numpy as jnp
from jax import lax
from jax.experimental import pallas as pl
from jax.experimental.pallas import tpu as pltpu
from jax.experimental.pallas import tpu_sc as plsc

_NUM_CORES = 2
_NUM_SUBCORES = 16
_NUM_WORKERS = _NUM_CORES * _NUM_SUBCORES
_LANES = 16
_PG = 4
_CG = 8
_P = 80
_NBUF = 4


@functools.partial(jax.jit, static_argnames=("pad",))
def _sc_lpe(xf, mk, idx, pe_table, pad):
    n = xf.shape[0]
    dc = xf.shape[2]
    d = _CG * dc
    v = pe_table.shape[1] // dc
    scale = math.sqrt(d)
    per_pg = n // _PG
    n_chunks = per_pg // _P
    assert n_chunks % _NBUF == 0
    mesh = plsc.VectorSubcoreMesh(core_axis_name="c", subcore_axis_name="s")

    @functools.partial(
        pl.kernel,
        mesh=mesh,
        out_type=jax.ShapeDtypeStruct((n, _CG, dc), jnp.float32),
        compiler_params=pltpu.CompilerParams(needs_layout_passes=False),
        scratch_types=[
            pltpu.VMEM((v * dc,), jnp.float32),
            *[pltpu.VMEM((_P,), jnp.int32) for _ in range(_NBUF)],
            *[pltpu.VMEM((_P,), jnp.int32) for _ in range(_NBUF)],
            *[pltpu.VMEM((_P, dc), jnp.float32) for _ in range(_NBUF)],
            *[pltpu.SemaphoreType.DMA for _ in range(2 * _NBUF)],
        ],
    )
    def k(x_hbm, mk_hbm, idx_hbm, tab_hbm, out_hbm, tab_v, *bufs):
        idxb = bufs[0:_NBUF]
        mkb = bufs[_NBUF : 2 * _NBUF]
        xb = bufs[2 * _NBUF : 3 * _NBUF]
        sin = bufs[3 * _NBUF : 4 * _NBUF]
        sout = bufs[4 * _NBUF : 5 * _NBUF]
        wid = lax.axis_index("s") * _NUM_CORES + lax.axis_index("c")
        pg = wid % _PG
        cg = wid // _PG
        col0 = cg * dc
        pos0 = pg * per_pg

        pltpu.sync_copy(tab_hbm.at[cg], tab_v)

        def issue_in(c, b):
            base = pos0 + c * _P
            pltpu.async_copy(idx_hbm.at[pl.ds(base, _P)], idxb[b], sin[b])
            pltpu.async_copy(mk_hbm.at[pl.ds(base, _P)], mkb[b], sin[b])
            pltpu.async_copy(
                x_hbm.at[pl.ds(base, _P), cg, :], xb[b], sin[b]
            )

        def wait_in(c, b):
            base = pos0 + c * _P
            pltpu.make_async_copy(idx_hbm.at[pl.ds(base, _P)], idxb[b], sin[b]).wait()
            pltpu.make_async_copy(mk_hbm.at[pl.ds(base, _P)], mkb[b], sin[b]).wait()
            pltpu.make_async_copy(
                x_hbm.at[pl.ds(base, _P), cg, :], xb[b], sin[b]
            ).wait()

        def issue_out(c, b):
            base = pos0 + c * _P
            pltpu.async_copy(
                xb[b], out_hbm.at[pl.ds(base, _P), cg, :], sout[b]
            )

        def wait_out(c, b):
            base = pos0 + c * _P
            pltpu.make_async_copy(
                xb[b], out_hbm.at[pl.ds(base, _P), cg, :], sout[b]
            ).wait()

        iota16 = lax.iota(jnp.int32, _LANES)

        for b in range(_NBUF - 1):
            issue_in(b, b)

        @pl.loop(0, n_chunks, step=_NBUF)
        def _main(g):
            for b in range(_NBUF):
                c = g + b
                wait_in(c, b)

                def _pad(i, carry):
                    sl = pl.ds(i * _LANES, _LANES)
                    idxb[b][sl] = jnp.where(
                        mkb[b][sl] != 0, pad, jnp.minimum(idxb[b][sl], pad)
                    )
                    return carry

                lax.fori_loop(0, _P // _LANES, _pad, 0)

                def _gfma(g2, carry):
                    pb = g2 * _LANES
                    vec = idxb[b][pl.ds(pb, _LANES)]
                    for l in range(_LANES):
                        r = vec[l]
                        p = pb + l
                        xb[b][p, :] = (
                            scale * xb[b][p, :] + tab_v[pl.ds(r * dc, dc)]
                        )
                    return carry

                lax.fori_loop(0, _P // _LANES, _gfma, 0)

                issue_out(c, b)
                nxt = c + _NBUF - 1
                bp = (b + _NBUF - 1) % _NBUF

                @pl.when(nxt < n_chunks)
                def _():
                    @pl.when(c >= 1)
                    def _():
                        wait_out(c - 1, bp)

                    issue_in(nxt, bp)

        for b in range(_NBUF):
            wait_out(n_chunks - _NBUF + b, b)

    return k(xf, mk, idx, pe_table)


def kernel(x, mask, indices, pe_table):
    b, s, d = x.shape
    n = b * s
    v = pe_table.shape[0]
    dc = d // _CG
    xf = x.reshape(n, _CG, dc)
    tab = pe_table.reshape(v, _CG, dc).transpose(1, 0, 2).reshape(_CG, v * dc)
    mk = mask.reshape(n).astype(jnp.int32)
    idx = indices.reshape(n).astype(jnp.int32)
    out = _sc_lpe(xf, mk, idx, tab, v - 1)
    return out.reshape(b, s, d)


# speedup vs baseline: 1.6789x; 1.1348x over previous
import functools
import math

import jax
import jax.numpy as jnp
from jax import lax
from jax.experimental import pallas as pl
from jax.experimental.pallas import tpu as pltpu
from jax.experimental.pallas import tpu_sc as plsc

_NUM_CORES = 2
_NUM_SUBCORES = 16
_NUM_WORKERS = _NUM_CORES * _NUM_SUBCORES
_LANES = 16
_PG = 8
_CG = 4
_P = 64
_NBUF = 4


@functools.partial(jax.jit, static_argnames=("pad",))
def _sc_lpe(xf, mk, idx, pe_table, pad):
    n = xf.shape[0]
    dc = xf.shape[2]
    d = _CG * dc
    v = pe_table.shape[1] // (dc // 2)
    scale = math.sqrt(d)
    per_pg = n // _PG
    n_chunks = per_pg // _P
    assert n_chunks % _NBUF == 0
    mesh = plsc.VectorSubcoreMesh(core_axis_name="c", subcore_axis_name="s")

    @functools.partial(
        pl.kernel,
        mesh=mesh,
        out_type=jax.ShapeDtypeStruct((n, _CG, dc), jnp.float32),
        compiler_params=pltpu.CompilerParams(needs_layout_passes=False),
        scratch_types=[
            pltpu.VMEM((v * dc // 2,), jnp.int32),
            *[pltpu.VMEM((_P,), jnp.int32) for _ in range(_NBUF)],
            *[pltpu.VMEM((_P,), jnp.int32) for _ in range(_NBUF)],
            *[pltpu.VMEM((_P, dc), jnp.float32) for _ in range(_NBUF)],
            *[pltpu.SemaphoreType.DMA for _ in range(2 * _NBUF)],
        ],
    )
    def k(x_hbm, mk_hbm, idx_hbm, tab_hbm, out_hbm, tab_v, *bufs):
        idxb = bufs[0:_NBUF]
        mkb = bufs[_NBUF : 2 * _NBUF]
        xb = bufs[2 * _NBUF : 3 * _NBUF]
        sin = bufs[3 * _NBUF : 4 * _NBUF]
        sout = bufs[4 * _NBUF : 5 * _NBUF]
        wid = lax.axis_index("s") * _NUM_CORES + lax.axis_index("c")
        pg = wid % _PG
        cg = wid // _PG
        col0 = cg * dc
        pos0 = pg * per_pg

        pltpu.sync_copy(tab_hbm.at[cg], tab_v)

        def issue_in(c, b):
            base = pos0 + c * _P
            pltpu.async_copy(idx_hbm.at[pl.ds(base, _P)], idxb[b], sin[b])
            pltpu.async_copy(mk_hbm.at[pl.ds(base, _P)], mkb[b], sin[b])
            pltpu.async_copy(
                x_hbm.at[pl.ds(base, _P), cg, :], xb[b], sin[b]
            )

        def wait_in(c, b):
            base = pos0 + c * _P
            pltpu.make_async_copy(idx_hbm.at[pl.ds(base, _P)], idxb[b], sin[b]).wait()
            pltpu.make_async_copy(mk_hbm.at[pl.ds(base, _P)], mkb[b], sin[b]).wait()
            pltpu.make_async_copy(
                x_hbm.at[pl.ds(base, _P), cg, :], xb[b], sin[b]
            ).wait()

        def issue_out(c, b):
            base = pos0 + c * _P
            pltpu.async_copy(
                xb[b], out_hbm.at[pl.ds(base, _P), cg, :], sout[b]
            )

        def wait_out(c, b):
            base = pos0 + c * _P
            pltpu.make_async_copy(
                xb[b], out_hbm.at[pl.ds(base, _P), cg, :], sout[b]
            ).wait()

        iota16 = lax.iota(jnp.int32, _LANES)

        for b in range(_NBUF - 1):
            issue_in(b, b)

        @pl.loop(0, n_chunks, step=_NBUF)
        def _main(g):
            for b in range(_NBUF):
                c = g + b
                wait_in(c, b)

                def _pad(i, carry):
                    sl = pl.ds(i * _LANES, _LANES)
                    idxb[b][sl] = jnp.where(
                        mkb[b][sl] != 0, pad, jnp.minimum(idxb[b][sl], pad)
                    )
                    return carry

                lax.fori_loop(0, _P // _LANES, _pad, 0)

                def _gfma(g2, carry):
                    pb = g2 * _LANES
                    vec = idxb[b][pl.ds(pb, _LANES)]
                    hw = dc // 2
                    for l in range(_LANES):
                        r = vec[l]
                        p = pb + l
                        w = tab_v[pl.ds(r * hw, hw)]
                        bf = plsc.bitcast(w, jnp.bfloat16)
                        ta, tb = plsc.unpack(
                            bf,
                            format=plsc.PackFormat.INTERLEAVED,
                            preferred_element_type=jnp.float32,
                        )
                        sl_a = pl.ds(0, hw)
                        sl_b = pl.ds(hw, hw)
                        xb[b][p, sl_a] = scale * xb[b][p, sl_a] + ta
                        xb[b][p, sl_b] = scale * xb[b][p, sl_b] + tb
                    return carry

                lax.fori_loop(0, _P // _LANES, _gfma, 0)

                issue_out(c, b)
                nxt = c + _NBUF - 1
                bp = (b + _NBUF - 1) % _NBUF

                @pl.when(nxt < n_chunks)
                def _():
                    @pl.when(c >= 1)
                    def _():
                        wait_out(c - 1, bp)

                    issue_in(nxt, bp)

        for b in range(_NBUF):
            wait_out(n_chunks - _NBUF + b, b)

    return k(xf, mk, idx, pe_table)


def kernel(x, mask, indices, pe_table):
    b, s, d = x.shape
    n = b * s
    v = pe_table.shape[0]
    dc = d // _CG
    xf = x.reshape(n, _CG, dc)
    hw = dc // 2
    perm = jnp.stack(
        [jnp.arange(hw), jnp.arange(hw) + hw], axis=1
    ).reshape(-1)
    tab_bf = pe_table.astype(jnp.bfloat16).reshape(v, _CG, dc)[:, :, perm]
    tab = jax.lax.bitcast_convert_type(
        tab_bf.transpose(1, 0, 2).reshape(_CG, v * hw, 2), jnp.int32
    )
    mk = mask.reshape(n).astype(jnp.int32)
    idx = indices.reshape(n).astype(jnp.int32)
    out = _sc_lpe(xf, mk, idx, tab, v - 1)
    return out.reshape(b, s, d)
